# baseline (speedup 1.0000x reference)
import jax
import jax.numpy as jnp
from jax import lax
from jax.experimental import pallas as pl
from jax.experimental.pallas import tpu as pltpu
from jax.experimental.pallas import tpu_sc as plsc

N_NODES = 10000
N_EDGES = 320000
D_IN = 128
D_H = 128
N_CLASSES = 64

NC = 2
NS = 16

EPT = N_EDGES // NS
K = 125
NCH = EPT // K
NBUF = 4
RPT = N_NODES // NS
HW = 16


def _make_spmm(D2):
  lanes_per_row = D2 // 16
  mesh = plsc.VectorSubcoreMesh(core_axis_name="c", subcore_axis_name="s")

  def body(x0_hbm, x1_hbm, src_hbm, dst_hbm, out_hbm,
           sv, dv, acc, r0, r1, r2, r3, gsemA, gsemB, ssemA, ssemB):
    cid = lax.axis_index("c")
    sid = lax.axis_index("s")
    pltpu.sync_copy(src_hbm.at[sid], sv)

    @pl.when(cid == 0)
    def _():
      pltpu.async_copy(x0_hbm.at[sv.at[0]], r2, gsemA)
      pltpu.async_copy(x0_hbm.at[sv.at[1]], r3, gsemA)

    @pl.when(cid == 1)
    def _():
      pltpu.async_copy(x1_hbm.at[sv.at[0]], r2, gsemA)
      pltpu.async_copy(x1_hbm.at[sv.at[1]], r3, gsemA)

    pltpu.sync_copy(dst_hbm.at[sid], dv)

    @pl.loop(0, K)
    def _zero(i):
      for j in range(lanes_per_row):
        r0[i, pl.ds(j * 16, 16)] = jnp.zeros((16,), jnp.float32)

    for r in range(RPT // K):
      pltpu.sync_copy(r0, acc.at[pl.ds(sid * RPT + r * K, K)])
    plsc.subcore_barrier()

    def edge_loop(x_hbm):
      def gather(chunk, buf, sem):
        pltpu.async_copy(x_hbm.at[sv.at[chunk]], buf, sem)

      def wait_gather(buf, sem):
        pltpu.make_async_copy(x_hbm.at[sv.at[0]], buf, sem).wait()

      def scatter(chunk, buf, sem):
        pltpu.async_copy(buf, acc.at[dv.at[chunk]], sem, add=True)

      def drain_scatter(buf, sem):
        pltpu.make_async_copy(buf, acc.at[dv.at[0]], sem).wait()

      @pl.loop(0, NCH, step=4)
      def _grp(g):
        @pl.when(g > 0)
        def _():
          drain_scatter(r0, ssemB)
          drain_scatter(r1, ssemB)

        gather(g + 2, r0, gsemB)
        gather(g + 3, r1, gsemB)
        wait_gather(r2, gsemA)
        wait_gather(r3, gsemA)
        scatter(g, r2, ssemA)
        scatter(g + 1, r3, ssemA)
        drain_scatter(r2, ssemA)
        drain_scatter(r3, ssemA)

        @pl.when(g + 4 < NCH)
        def _():
          gather(g + 4, r2, gsemA)
          gather(g + 5, r3, gsemA)

        wait_gather(r0, gsemB)
        wait_gather(r1, gsemB)
        scatter(g + 2, r0, ssemB)
        scatter(g + 3, r1, ssemB)

      drain_scatter(r0, ssemB)
      drain_scatter(r1, ssemB)

    @pl.when(cid == 0)
    def _():
      edge_loop(x0_hbm)

    @pl.when(cid == 1)
    def _():
      edge_loop(x1_hbm)

    plsc.subcore_barrier()
    pltpu.sync_copy(acc.at[pl.ds(sid * RPT, RPT)],
                    out_hbm.at[cid, pl.ds(sid * RPT, RPT)])

  return pl.kernel(
      body,
      out_type=jax.ShapeDtypeStruct((NC, N_NODES, D2), jnp.float32),
      mesh=mesh,
      compiler_params=pltpu.CompilerParams(use_tc_tiling_on_sc=False, skip_device_barrier=True),
      scratch_types=[
          pltpu.VMEM((NCH, K), jnp.int32),
          pltpu.VMEM((NCH, K), jnp.int32),
          pltpu.VMEM_SHARED((N_NODES, D2), jnp.float32),
          pltpu.VMEM((K, D2), jnp.float32),
          pltpu.VMEM((K, D2), jnp.float32),
          pltpu.VMEM((K, D2), jnp.float32),
          pltpu.VMEM((K, D2), jnp.float32),
          pltpu.SemaphoreType.DMA,
          pltpu.SemaphoreType.DMA,
          pltpu.SemaphoreType.DMA,
          pltpu.SemaphoreType.DMA,
      ],
  )


def _make_deg():
  mesh = plsc.VectorSubcoreMesh(core_axis_name="c", subcore_axis_name="s")

  def body(eidx_hbm, out_hbm, iv, acc, ones, sem):
    cid = lax.axis_index("c")
    sid = lax.axis_index("s")
    pltpu.sync_copy(eidx_hbm.at[cid, sid], iv)

    @pl.loop(0, K)
    def _fill(i):
      ones[i, pl.ds(0, 16)] = jnp.zeros((16,), jnp.float32)

    for r in range(RPT // K):
      pltpu.sync_copy(ones, acc.at[pl.ds(sid * RPT + r * K, K)])

    @pl.loop(0, K)
    def _fill2(i):
      ones[i, pl.ds(0, 16)] = jnp.full((16,), 1.0, jnp.float32)

    plsc.subcore_barrier()

    @pl.loop(0, NCH, step=2 * NBUF)
    def _grp(g):
      puts = [pltpu.async_copy(ones, acc.at[iv.at[g + b]], sem, add=True)
              for b in range(2 * NBUF)]
      for c in puts:
        c.wait()

    plsc.subcore_barrier()
    pltpu.sync_copy(acc.at[pl.ds(sid * RPT, RPT)],
                    out_hbm.at[cid, pl.ds(sid * RPT, RPT)])

  return pl.kernel(
      body,
      out_type=jax.ShapeDtypeStruct((NC, N_NODES, HW), jnp.float32),
      mesh=mesh,
      compiler_params=pltpu.CompilerParams(use_tc_tiling_on_sc=False, skip_device_barrier=True),
      scratch_types=[
          pltpu.VMEM((NCH, K), jnp.int32),
          pltpu.VMEM_SHARED((N_NODES, HW), jnp.float32),
          pltpu.VMEM((K, HW), jnp.float32),
          pltpu.SemaphoreType.DMA,
      ],
  )


_spmm64 = _make_spmm(D_H // 2)
_spmm32 = _make_spmm(N_CLASSES // 2)
_deg = _make_deg()

_BR = 1000


def _tc_proj(F, Ws, bs, hist):
  Dh = D_H // 2

  def body(f_ref, w_ref, b_ref, d_ref, o0_ref, o1_ref):
    h = jnp.dot(f_ref[0], w_ref[0], preferred_element_type=jnp.float32)
    h = h + b_ref[0]
    norm = lax.rsqrt(jnp.maximum(d_ref[0, :, 0:1], 1.0))
    h = h * norm
    o0_ref[...] = h[:, :Dh]
    o1_ref[...] = h[:, Dh:]

  return pl.pallas_call(
      body,
      grid=(2, 5000 // _BR),
      in_specs=[
          pl.BlockSpec((1, _BR, D_IN), lambda i, j: (i, j, 0)),
          pl.BlockSpec((1, D_IN, D_H), lambda i, j: (i, 0, 0)),
          pl.BlockSpec((1, 1, D_H), lambda i, j: (i, 0, 0)),
          pl.BlockSpec((1, _BR, HW), lambda i, j: (0, i * (5000 // _BR) + j, 0)),
      ],
      out_specs=[
          pl.BlockSpec((_BR, Dh), lambda i, j: (i * (5000 // _BR) + j, 0)),
          pl.BlockSpec((_BR, Dh), lambda i, j: (i * (5000 // _BR) + j, 0)),
      ],
      out_shape=[
          jax.ShapeDtypeStruct((N_NODES, Dh), jnp.float32),
          jax.ShapeDtypeStruct((N_NODES, Dh), jnp.float32),
      ],
  )(F, Ws, bs, hist)


def _tc_mid(p, hist, b2, W2, emit_h2=False):
  Dp2 = p.shape[2]
  Do = W2.shape[2]
  Do2 = Do // 2

  def body(p_ref, d_ref, b_ref, w_ref, *out_refs):
    ndst = lax.rsqrt(jnp.maximum(d_ref[1, :, 0:1], 1.0))
    ha = jnp.maximum(p_ref[0] * ndst + b_ref[0], 0.0)
    hb = jnp.maximum(p_ref[1] * ndst + b_ref[1], 0.0)
    m = (jnp.dot(ha, w_ref[0], preferred_element_type=jnp.float32)
         + jnp.dot(hb, w_ref[1], preferred_element_type=jnp.float32))
    nsrc = lax.rsqrt(jnp.maximum(d_ref[0, :, 0:1], 1.0))
    m = m * nsrc
    out_refs[0][...] = m[:, :Do2]
    out_refs[1][...] = m[:, Do2:]
    if emit_h2:
      out_refs[2][:, :Dp2] = ha
      out_refs[2][:, Dp2:] = hb

  out_specs = [
      pl.BlockSpec((_BR, Do2), lambda j: (j, 0)),
      pl.BlockSpec((_BR, Do2), lambda j: (j, 0)),
  ]
  out_shape = [
      jax.ShapeDtypeStruct((N_NODES, Do2), jnp.float32),
      jax.ShapeDtypeStruct((N_NODES, Do2), jnp.float32),
  ]
  if emit_h2:
    out_specs.append(pl.BlockSpec((_BR, 2 * Dp2), lambda j: (j, 0)))
    out_shape.append(jax.ShapeDtypeStruct((N_NODES, 2 * Dp2), jnp.float32))

  return pl.pallas_call(
      body,
      grid=(N_NODES // _BR,),
      in_specs=[
          pl.BlockSpec((2, _BR, Dp2), lambda j: (0, j, 0)),
          pl.BlockSpec((2, _BR, HW), lambda j: (0, j, 0)),
          pl.BlockSpec((2, 1, Dp2), lambda j: (0, 0, 0)),
          pl.BlockSpec((2, Dp2, Do), lambda j: (0, 0, 0)),
      ],
      out_specs=out_specs,
      out_shape=out_shape,
  )(p, hist, b2, W2)


def _tc_final(p, hist, b2):
  Dp2 = p.shape[2]

  def body(p_ref, d_ref, b_ref, o_ref):
    ndst = lax.rsqrt(jnp.maximum(d_ref[0, :, 0:1], 1.0))
    o_ref[:, :Dp2] = p_ref[0] * ndst + b_ref[0]
    o_ref[:, Dp2:] = p_ref[1] * ndst + b_ref[1]

  return pl.pallas_call(
      body,
      grid=(N_NODES // _BR,),
      in_specs=[
          pl.BlockSpec((2, _BR, Dp2), lambda j: (0, j, 0)),
          pl.BlockSpec((1, _BR, HW), lambda j: (1, j, 0)),
          pl.BlockSpec((2, 1, Dp2), lambda j: (0, 0, 0)),
      ],
      out_specs=pl.BlockSpec((_BR, 2 * Dp2), lambda j: (j, 0)),
      out_shape=jax.ShapeDtypeStruct((N_NODES, 2 * Dp2), jnp.float32),
  )(p, hist, b2)


def _halves(v):
  if v.ndim == 1:
    return v.reshape(2, 1, v.shape[0] // 2)
  return v.reshape(2, v.shape[0] // 2, v.shape[1])


def kernel(feat0, feat1, e_feat, edge_index, W_fc0, b_fc0, W_fc1, b_fc1,
           b_gc0, W_gc1, b_gc1, W_gc2, b_gc2):
  del e_feat

  eidx_r = edge_index.reshape(2, NS, NCH, K)
  hist = _deg(eidx_r)

  F = jnp.stack([feat0, feat1])
  Ws = jnp.stack([W_fc0, W_fc1])
  bs = jnp.stack([b_fc0, b_fc1])[:, None, :]
  m0a, m0b = _tc_proj(F, Ws, bs, hist)

  srcr = edge_index[0].reshape(NS, NCH, K)
  dstr = edge_index[1].reshape(NS, NCH, K)

  p0 = _spmm64(m0a, m0b, srcr, dstr)
  m1a, m1b = _tc_mid(p0, hist, _halves(b_gc0), _halves(W_gc1))
  p1 = _spmm64(m1a, m1b, srcr, dstr)
  m2a, m2b, h2 = _tc_mid(p1, hist, _halves(b_gc1),
                         _halves(W_gc2), emit_h2=True)
  p2 = _spmm32(m2a, m2b, srcr, dstr)
  h3 = _tc_final(p2, hist, _halves(b_gc2))
  return (h3, h2)


# speedup vs baseline: 1.0215x; 1.0215x over previous
import jax
import jax.numpy as jnp
from jax import lax
from jax.experimental import pallas as pl
from jax.experimental.pallas import tpu as pltpu
from jax.experimental.pallas import tpu_sc as plsc

N_NODES = 10000
N_EDGES = 320000
D_IN = 128
D_H = 128
N_CLASSES = 64

NC = 2
NS = 16

EPT = N_EDGES // NS
K = 125
NCH = EPT // K
NBUF = 4
RPT = N_NODES // NS
HW = 16


def _make_spmm(D2):
  lanes_per_row = D2 // 16
  mesh = plsc.VectorSubcoreMesh(core_axis_name="c", subcore_axis_name="s")

  def body(x0_hbm, x1_hbm, src_hbm, dst_hbm, out_hbm,
           sv, dv, acc, r0, r1, r2, r3, gsemA, gsemB, ssemA, ssemB):
    cid = lax.axis_index("c")
    sid = lax.axis_index("s")
    pltpu.sync_copy(src_hbm.at[sid], sv)

    @pl.when(cid == 0)
    def _():
      pltpu.async_copy(x0_hbm.at[sv.at[0]], r2, gsemA)
      pltpu.async_copy(x0_hbm.at[sv.at[1]], r3, gsemA)

    @pl.when(cid == 1)
    def _():
      pltpu.async_copy(x1_hbm.at[sv.at[0]], r2, gsemA)
      pltpu.async_copy(x1_hbm.at[sv.at[1]], r3, gsemA)

    pltpu.sync_copy(dst_hbm.at[sid], dv)

    @pl.loop(0, K)
    def _zero(i):
      for j in range(lanes_per_row):
        r0[i, pl.ds(j * 16, 16)] = jnp.zeros((16,), jnp.float32)

    for r in range(RPT // K):
      pltpu.sync_copy(r0, acc.at[pl.ds(sid * RPT + r * K, K)])
    plsc.subcore_barrier()

    def edge_loop(x_hbm):
      def gather(chunk, buf, sem):
        pltpu.async_copy(x_hbm.at[sv.at[chunk]], buf, sem)

      def wait_gather(buf, sem):
        pltpu.make_async_copy(x_hbm.at[sv.at[0]], buf, sem).wait()

      def scatter(chunk, buf, sem):
        pltpu.async_copy(buf, acc.at[dv.at[chunk]], sem, add=True)

      def drain_scatter(buf, sem):
        pltpu.make_async_copy(buf, acc.at[dv.at[0]], sem).wait()

      @pl.loop(0, NCH, step=4)
      def _grp(g):
        @pl.when(g > 0)
        def _():
          drain_scatter(r0, ssemB)
          drain_scatter(r1, ssemB)

        gather(g + 2, r0, gsemB)
        gather(g + 3, r1, gsemB)
        wait_gather(r2, gsemA)
        wait_gather(r3, gsemA)
        scatter(g, r2, ssemA)
        scatter(g + 1, r3, ssemA)
        drain_scatter(r2, ssemA)
        drain_scatter(r3, ssemA)

        @pl.when(g + 4 < NCH)
        def _():
          gather(g + 4, r2, gsemA)
          gather(g + 5, r3, gsemA)

        wait_gather(r0, gsemB)
        wait_gather(r1, gsemB)
        scatter(g + 2, r0, ssemB)
        scatter(g + 3, r1, ssemB)

      drain_scatter(r0, ssemB)
      drain_scatter(r1, ssemB)

    @pl.when(cid == 0)
    def _():
      edge_loop(x0_hbm)

    @pl.when(cid == 1)
    def _():
      edge_loop(x1_hbm)

    plsc.subcore_barrier()
    pltpu.sync_copy(acc.at[pl.ds(sid * RPT, RPT)],
                    out_hbm.at[cid, pl.ds(sid * RPT, RPT)])

  return pl.kernel(
      body,
      out_type=jax.ShapeDtypeStruct((NC, N_NODES, D2), jnp.float32),
      mesh=mesh,
      compiler_params=pltpu.CompilerParams(use_tc_tiling_on_sc=False),
      scratch_types=[
          pltpu.VMEM((NCH, K), jnp.int32),
          pltpu.VMEM((NCH, K), jnp.int32),
          pltpu.VMEM_SHARED((N_NODES, D2), jnp.float32),
          pltpu.VMEM((K, D2), jnp.float32),
          pltpu.VMEM((K, D2), jnp.float32),
          pltpu.VMEM((K, D2), jnp.float32),
          pltpu.VMEM((K, D2), jnp.float32),
          pltpu.SemaphoreType.DMA,
          pltpu.SemaphoreType.DMA,
          pltpu.SemaphoreType.DMA,
          pltpu.SemaphoreType.DMA,
      ],
  )


ES_NCH = 80


def _make_spmm_es(D):
  lanes_per_row = D // 16
  mesh = plsc.VectorSubcoreMesh(core_axis_name="c", subcore_axis_name="s")

  def body(x_hbm, src_hbm, dst_hbm, out_hbm,
           sv, dv, acc, r0, r1, r2, r3, gsemA, gsemB, ssemA, ssemB):
    cid = lax.axis_index("c")
    sid = lax.axis_index("s")
    pltpu.sync_copy(src_hbm.at[cid, sid], sv)

    pltpu.async_copy(x_hbm.at[sv.at[0]], r2, gsemA)
    pltpu.async_copy(x_hbm.at[sv.at[1]], r3, gsemA)

    pltpu.sync_copy(dst_hbm.at[cid, sid], dv)

    @pl.loop(0, K)
    def _zero(i):
      for j in range(lanes_per_row):
        r0[i, pl.ds(j * 16, 16)] = jnp.zeros((16,), jnp.float32)

    for r in range(RPT // K):
      pltpu.sync_copy(r0, acc.at[pl.ds(sid * RPT + r * K, K)])
    plsc.subcore_barrier()

    def gather(chunk, buf, sem):
      pltpu.async_copy(x_hbm.at[sv.at[chunk]], buf, sem)

    def wait_gather(buf, sem):
      pltpu.make_async_copy(x_hbm.at[sv.at[0]], buf, sem).wait()

    def scatter(chunk, buf, sem):
      pltpu.async_copy(buf, acc.at[dv.at[chunk]], sem, add=True)

    def drain_scatter(buf, sem):
      pltpu.make_async_copy(buf, acc.at[dv.at[0]], sem).wait()

    @pl.loop(0, ES_NCH, step=4)
    def _grp(g):
      @pl.when(g > 0)
      def _():
        drain_scatter(r0, ssemB)
        drain_scatter(r1, ssemB)

      gather(g + 2, r0, gsemB)
      gather(g + 3, r1, gsemB)
      wait_gather(r2, gsemA)
      wait_gather(r3, gsemA)
      scatter(g, r2, ssemA)
      scatter(g + 1, r3, ssemA)
      drain_scatter(r2, ssemA)
      drain_scatter(r3, ssemA)

      @pl.when(g + 4 < ES_NCH)
      def _():
        gather(g + 4, r2, gsemA)
        gather(g + 5, r3, gsemA)

      wait_gather(r0, gsemB)
      wait_gather(r1, gsemB)
      scatter(g + 2, r0, ssemB)
      scatter(g + 3, r1, ssemB)

    drain_scatter(r0, ssemB)
    drain_scatter(r1, ssemB)

    plsc.subcore_barrier()
    pltpu.sync_copy(acc.at[pl.ds(sid * RPT, RPT)],
                    out_hbm.at[cid, pl.ds(sid * RPT, RPT)])

  return pl.kernel(
      body,
      out_type=jax.ShapeDtypeStruct((NC, N_NODES, D), jnp.float32),
      mesh=mesh,
      compiler_params=pltpu.CompilerParams(use_tc_tiling_on_sc=False),
      scratch_types=[
          pltpu.VMEM((ES_NCH, K), jnp.int32),
          pltpu.VMEM((ES_NCH, K), jnp.int32),
          pltpu.VMEM_SHARED((N_NODES, D), jnp.float32),
          pltpu.VMEM((K, D), jnp.float32),
          pltpu.VMEM((K, D), jnp.float32),
          pltpu.VMEM((K, D), jnp.float32),
          pltpu.VMEM((K, D), jnp.float32),
          pltpu.SemaphoreType.DMA,
          pltpu.SemaphoreType.DMA,
          pltpu.SemaphoreType.DMA,
          pltpu.SemaphoreType.DMA,
      ],
  )


def _make_deg():
  mesh = plsc.VectorSubcoreMesh(core_axis_name="c", subcore_axis_name="s")

  def body(eidx_hbm, out_hbm, iv, acc, ones, sem):
    cid = lax.axis_index("c")
    sid = lax.axis_index("s")
    pltpu.sync_copy(eidx_hbm.at[cid, sid], iv)

    @pl.loop(0, K)
    def _fill(i):
      ones[i, pl.ds(0, 16)] = jnp.zeros((16,), jnp.float32)

    for r in range(RPT // K):
      pltpu.sync_copy(ones, acc.at[pl.ds(sid * RPT + r * K, K)])

    @pl.loop(0, K)
    def _fill2(i):
      ones[i, pl.ds(0, 16)] = jnp.full((16,), 1.0, jnp.float32)

    plsc.subcore_barrier()

    @pl.loop(0, NCH, step=2 * NBUF)
    def _grp(g):
      puts = [pltpu.async_copy(ones, acc.at[iv.at[g + b]], sem, add=True)
              for b in range(2 * NBUF)]
      for c in puts:
        c.wait()

    plsc.subcore_barrier()
    pltpu.sync_copy(acc.at[pl.ds(sid * RPT, RPT)],
                    out_hbm.at[cid, pl.ds(sid * RPT, RPT)])

  return pl.kernel(
      body,
      out_type=jax.ShapeDtypeStruct((NC, N_NODES, HW), jnp.float32),
      mesh=mesh,
      compiler_params=pltpu.CompilerParams(use_tc_tiling_on_sc=False),
      scratch_types=[
          pltpu.VMEM((NCH, K), jnp.int32),
          pltpu.VMEM_SHARED((N_NODES, HW), jnp.float32),
          pltpu.VMEM((K, HW), jnp.float32),
          pltpu.SemaphoreType.DMA,
      ],
  )


_spmm64 = _make_spmm(D_H // 2)
_spmm64es = _make_spmm_es(N_CLASSES)
_deg = _make_deg()

_BR = 1000


def _tc_proj(F, Ws, bs, hist):
  Dh = D_H // 2

  def body(f_ref, w_ref, b_ref, d_ref, o0_ref, o1_ref):
    h = jnp.dot(f_ref[0], w_ref[0], preferred_element_type=jnp.float32)
    h = h + b_ref[0]
    norm = lax.rsqrt(jnp.maximum(d_ref[0, :, 0:1], 1.0))
    h = h * norm
    o0_ref[...] = h[:, :Dh]
    o1_ref[...] = h[:, Dh:]

  return pl.pallas_call(
      body,
      grid=(2, 5000 // _BR),
      in_specs=[
          pl.BlockSpec((1, _BR, D_IN), lambda i, j: (i, j, 0)),
          pl.BlockSpec((1, D_IN, D_H), lambda i, j: (i, 0, 0)),
          pl.BlockSpec((1, 1, D_H), lambda i, j: (i, 0, 0)),
          pl.BlockSpec((1, _BR, HW), lambda i, j: (0, i * (5000 // _BR) + j, 0)),
      ],
      out_specs=[
          pl.BlockSpec((_BR, Dh), lambda i, j: (i * (5000 // _BR) + j, 0)),
          pl.BlockSpec((_BR, Dh), lambda i, j: (i * (5000 // _BR) + j, 0)),
      ],
      out_shape=[
          jax.ShapeDtypeStruct((N_NODES, Dh), jnp.float32),
          jax.ShapeDtypeStruct((N_NODES, Dh), jnp.float32),
      ],
  )(F, Ws, bs, hist)


def _tc_mid(p, hist, b2, W2, emit_h2=False, split_out=True):
  Dp2 = p.shape[2]
  Do = W2.shape[2]
  Do2 = Do // 2

  def body(p_ref, d_ref, b_ref, w_ref, *out_refs):
    ndst = lax.rsqrt(jnp.maximum(d_ref[1, :, 0:1], 1.0))
    ha = jnp.maximum(p_ref[0] * ndst + b_ref[0], 0.0)
    hb = jnp.maximum(p_ref[1] * ndst + b_ref[1], 0.0)
    m = (jnp.dot(ha, w_ref[0], preferred_element_type=jnp.float32)
         + jnp.dot(hb, w_ref[1], preferred_element_type=jnp.float32))
    nsrc = lax.rsqrt(jnp.maximum(d_ref[0, :, 0:1], 1.0))
    m = m * nsrc
    if split_out:
      out_refs[0][...] = m[:, :Do2]
      out_refs[1][...] = m[:, Do2:]
    else:
      out_refs[0][...] = m
    if emit_h2:
      out_refs[-1][:, :Dp2] = ha
      out_refs[-1][:, Dp2:] = hb

  if split_out:
    out_specs = [
        pl.BlockSpec((_BR, Do2), lambda j: (j, 0)),
        pl.BlockSpec((_BR, Do2), lambda j: (j, 0)),
    ]
    out_shape = [
        jax.ShapeDtypeStruct((N_NODES, Do2), jnp.float32),
        jax.ShapeDtypeStruct((N_NODES, Do2), jnp.float32),
    ]
  else:
    out_specs = [pl.BlockSpec((_BR, Do), lambda j: (j, 0))]
    out_shape = [jax.ShapeDtypeStruct((N_NODES, Do), jnp.float32)]
  if emit_h2:
    out_specs.append(pl.BlockSpec((_BR, 2 * Dp2), lambda j: (j, 0)))
    out_shape.append(jax.ShapeDtypeStruct((N_NODES, 2 * Dp2), jnp.float32))

  return pl.pallas_call(
      body,
      grid=(N_NODES // _BR,),
      in_specs=[
          pl.BlockSpec((2, _BR, Dp2), lambda j: (0, j, 0)),
          pl.BlockSpec((2, _BR, HW), lambda j: (0, j, 0)),
          pl.BlockSpec((2, 1, Dp2), lambda j: (0, 0, 0)),
          pl.BlockSpec((2, Dp2, Do), lambda j: (0, 0, 0)),
      ],
      out_specs=out_specs,
      out_shape=out_shape,
  )(p, hist, b2, W2)


def _tc_final(p, hist, b):
  Dp = p.shape[2]

  def body(p_ref, d_ref, b_ref, o_ref):
    ndst = lax.rsqrt(jnp.maximum(d_ref[0, :, 0:1], 1.0))
    o_ref[...] = (p_ref[0] + p_ref[1]) * ndst + b_ref[...]

  return pl.pallas_call(
      body,
      grid=(N_NODES // _BR,),
      in_specs=[
          pl.BlockSpec((2, _BR, Dp), lambda j: (0, j, 0)),
          pl.BlockSpec((1, _BR, HW), lambda j: (1, j, 0)),
          pl.BlockSpec((1, Dp), lambda j: (0, 0)),
      ],
      out_specs=pl.BlockSpec((_BR, Dp), lambda j: (j, 0)),
      out_shape=jax.ShapeDtypeStruct((N_NODES, Dp), jnp.float32),
  )(p, hist, b)


def _halves(v):
  if v.ndim == 1:
    return v.reshape(2, 1, v.shape[0] // 2)
  return v.reshape(2, v.shape[0] // 2, v.shape[1])


def kernel(feat0, feat1, e_feat, edge_index, W_fc0, b_fc0, W_fc1, b_fc1,
           b_gc0, W_gc1, b_gc1, W_gc2, b_gc2):
  del e_feat

  eidx_r = edge_index.reshape(2, NS, NCH, K)
  hist = _deg(eidx_r)

  F = jnp.stack([feat0, feat1])
  Ws = jnp.stack([W_fc0, W_fc1])
  bs = jnp.stack([b_fc0, b_fc1])[:, None, :]
  m0a, m0b = _tc_proj(F, Ws, bs, hist)

  srcr = edge_index[0].reshape(NS, NCH, K)
  dstr = edge_index[1].reshape(NS, NCH, K)

  p0 = _spmm64(m0a, m0b, srcr, dstr)
  m1a, m1b = _tc_mid(p0, hist, _halves(b_gc0), _halves(W_gc1))
  p1 = _spmm64(m1a, m1b, srcr, dstr)
  m2, h2 = _tc_mid(p1, hist, _halves(b_gc1),
                   _halves(W_gc2), emit_h2=True, split_out=False)
  srcr2 = edge_index[0].reshape(NC, NS, ES_NCH, K)
  dstr2 = edge_index[1].reshape(NC, NS, ES_NCH, K)
  p2 = _spmm64es(m2, srcr2, dstr2)
  h3 = _tc_final(p2, hist, b_gc2[None, :])
  return (h3, h2)
